# Initial kernel scaffold; baseline (speedup 1.0000x reference)
import functools

import jax
import jax.numpy as jnp
from jax import lax
from jax.experimental import pallas as pl
from jax.experimental.pallas import tpu as pltpu
from jax.experimental.pallas import tpu_sc as plsc

_VOCAB = 100000
_NE = 768
_T = 2048
_LANES = 16
_NC = 2
_NS = 16
_NW = _NC * _NS
_ROWS = 8192
_ROWS_PER_W = _ROWS // _NW
_CHUNK = 64
_NCHUNK = _ROWS_PER_W // _CHUNK


def _emb_body(idx_hbm, tok_hbm, pos_hbm, out_hbm, idx_v, mask_v, rows_v, pos_v, sem):
    c = lax.axis_index("c")
    s = lax.axis_index("s")
    wid = s * _NC + c
    base = wid * _ROWS_PER_W
    t0 = base % _T

    pltpu.sync_copy(idx_hbm.at[pl.ds(base, _ROWS_PER_W)], idx_v)

    for j in range(_ROWS_PER_W // _LANES):
        sl = pl.ds(j * _LANES, _LANES)
        v = idx_v[sl]
        bad = jnp.logical_or(v < 1, v > _VOCAB)
        idx_v[sl] = jnp.where(bad, 0, v)
        mask_v[sl] = jnp.where(bad, 0.0, 1.0).astype(jnp.float32)

    for ck in range(_NCHUNK):
        gather = pltpu.async_copy(
            tok_hbm.at[idx_v.at[pl.ds(ck * _CHUNK, _CHUNK)]], rows_v, sem)
        pltpu.sync_copy(pos_hbm.at[pl.ds(t0 + ck * _CHUNK, _CHUNK)], pos_v)
        gather.wait()

        def row_body(r, carry, ck=ck):
            rid = jnp.full((_LANES,), ck * _CHUNK + r, dtype=jnp.int32)
            m16 = plsc.load_gather(mask_v, [rid])
            for j in range(_NE // _LANES):
                sl = pl.ds(j * _LANES, _LANES)
                rows_v[r, sl] = rows_v[r, sl] * m16 + pos_v[r, sl]
            return carry

        lax.fori_loop(0, _CHUNK, row_body, 0)
        pltpu.sync_copy(rows_v, out_hbm.at[pl.ds(base + ck * _CHUNK, _CHUNK)])


@jax.jit
def _emb_call(idx_flat, tok_emb, pos2d):
    mesh = plsc.VectorSubcoreMesh(core_axis_name="c", subcore_axis_name="s")
    kfn = pl.kernel(
        _emb_body,
        mesh=mesh,
        out_type=jax.ShapeDtypeStruct((_ROWS, _NE), jnp.float32),
        scratch_types=[
            pltpu.VMEM((_ROWS_PER_W,), jnp.int32),
            pltpu.VMEM((_ROWS_PER_W,), jnp.float32),
            pltpu.VMEM((_CHUNK, _NE), jnp.float32),
            pltpu.VMEM((_CHUNK, _NE), jnp.float32),
            pltpu.SemaphoreType.DMA,
        ],
    )
    return kfn(idx_flat, tok_emb, pos2d)


def kernel(idx, tok_emb, pos_emb):
    b, t = idx.shape
    out = _emb_call(idx.reshape(-1), tok_emb, pos_emb.reshape(-1, _NE)[:t])
    return out.reshape(b, t, _NE)


# speedup vs baseline: 3.6194x; 3.6194x over previous
import functools

import jax
import jax.numpy as jnp
from jax import lax
from jax.experimental import pallas as pl
from jax.experimental.pallas import tpu as pltpu
from jax.experimental.pallas import tpu_sc as plsc

_VOCAB = 100000
_NE = 768
_T = 2048
_LANES = 16
_NC = 2
_NS = 16
_NW = _NC * _NS
_ROWS = 8192
_ROWS_PER_W = _ROWS // _NW
_CHUNK = 64
_NCHUNK = _ROWS_PER_W // _CHUNK


def _emb_body(idx_hbm, tok_hbm, pos_hbm, out_hbm, idx_v, mask_v, rows_v, pos_v, sem):
    c = lax.axis_index("c")
    s = lax.axis_index("s")
    wid = s * _NC + c
    base = wid * _ROWS_PER_W
    t0 = base % _T

    pltpu.sync_copy(idx_hbm.at[pl.ds(base, _ROWS_PER_W)], idx_v)

    for j in range(_ROWS_PER_W // _LANES):
        sl = pl.ds(j * _LANES, _LANES)
        v = idx_v[sl]
        bad = jnp.logical_or(v < 1, v > _VOCAB)
        idx_v[sl] = jnp.where(bad, 0, v)
        mask_v[sl] = jnp.where(bad, 0.0, 1.0).astype(jnp.float32)

    for ck in range(_NCHUNK):
        gather = pltpu.async_copy(
            tok_hbm.at[idx_v.at[pl.ds(ck * _CHUNK, _CHUNK)]], rows_v, sem)
        pltpu.sync_copy(pos_hbm.at[pl.ds(t0 + ck * _CHUNK, _CHUNK)], pos_v)
        gather.wait()

        def row_body(r, carry, ck=ck):
            m = mask_v[pl.ds(ck * _CHUNK + r, _LANES)][0]
            for j in range(_NE // _LANES):
                sl = pl.ds(j * _LANES, _LANES)
                rows_v[r, sl] = rows_v[r, sl] * m + pos_v[r, sl]
            return carry

        lax.fori_loop(0, _CHUNK, row_body, 0)
        pltpu.sync_copy(rows_v, out_hbm.at[pl.ds(base + ck * _CHUNK, _CHUNK)])


@jax.jit
def _emb_call(idx_flat, tok_emb, pos2d):
    mesh = plsc.VectorSubcoreMesh(core_axis_name="c", subcore_axis_name="s")
    kfn = pl.kernel(
        _emb_body,
        mesh=mesh,
        out_type=jax.ShapeDtypeStruct((_ROWS, _NE), jnp.float32),
        scratch_types=[
            pltpu.VMEM((_ROWS_PER_W,), jnp.int32),
            pltpu.VMEM((_ROWS_PER_W + _LANES,), jnp.float32),
            pltpu.VMEM((_CHUNK, _NE), jnp.float32),
            pltpu.VMEM((_CHUNK, _NE), jnp.float32),
            pltpu.SemaphoreType.DMA,
        ],
    )
    return kfn(idx_flat, tok_emb, pos2d)


def kernel(idx, tok_emb, pos_emb):
    b, t = idx.shape
    out = _emb_call(idx.reshape(-1), tok_emb, pos_emb.reshape(-1, _NE)[:t])
    return out.reshape(b, t, _NE)


# speedup vs baseline: 4.2336x; 1.1697x over previous
import functools

import jax
import jax.numpy as jnp
from jax import lax
from jax.experimental import pallas as pl
from jax.experimental.pallas import tpu as pltpu
from jax.experimental.pallas import tpu_sc as plsc

_VOCAB = 100000
_NE = 768
_T = 2048
_LANES = 16
_NC = 2
_NS = 16
_NW = _NC * _NS
_ROWS = 8192
_ROWS_PER_W = _ROWS // _NW
_CHUNK = 32
_NCHUNK = _ROWS_PER_W // _CHUNK


def _emb_body(idx_hbm, tok_hbm, pos_hbm, out_hbm,
              idx_v, mask_v, rows0, rows1, pos0, pos1,
              gs0, gs1, ps0, ps1, os0, os1):
    rows = (rows0, rows1)
    pos = (pos0, pos1)
    gsem = (gs0, gs1)
    psem = (ps0, ps1)
    osem = (os0, os1)

    c = lax.axis_index("c")
    s = lax.axis_index("s")
    wid = s * _NC + c
    base = wid * _ROWS_PER_W
    t0 = base % _T

    pltpu.sync_copy(idx_hbm.at[pl.ds(base, _ROWS_PER_W)], idx_v)

    for j in range(_ROWS_PER_W // _LANES):
        sl = pl.ds(j * _LANES, _LANES)
        v = idx_v[sl]
        bad = jnp.logical_or(v < 1, v > _VOCAB)
        idx_v[sl] = jnp.where(bad, 0, v)
        mask_v[sl] = jnp.where(bad, 0.0, 1.0).astype(jnp.float32)

    def issue(ck):
        b = ck % 2
        g = pltpu.async_copy(
            tok_hbm.at[idx_v.at[pl.ds(ck * _CHUNK, _CHUNK)]], rows[b], gsem[b])
        p = pltpu.async_copy(
            pos_hbm.at[pl.ds(t0 + ck * _CHUNK, _CHUNK)], pos[b], psem[b])
        return g, p

    inflight = {0: issue(0)}
    outcp = {}
    for ck in range(_NCHUNK):
        b = ck % 2
        if ck + 1 < _NCHUNK:
            if ck - 1 >= 0:
                outcp[ck - 1].wait()
            inflight[ck + 1] = issue(ck + 1)
        g, p = inflight[ck]
        g.wait()
        p.wait()

        def row_body(r, carry, ck=ck, b=b):
            m = mask_v[pl.ds(ck * _CHUNK + r, _LANES)][0]
            for j in range(_NE // _LANES):
                sl = pl.ds(j * _LANES, _LANES)
                rows[b][r, sl] = rows[b][r, sl] * m + pos[b][r, sl]
            return carry

        lax.fori_loop(0, _CHUNK, row_body, 0)
        outcp[ck] = pltpu.async_copy(
            rows[b], out_hbm.at[pl.ds(base + ck * _CHUNK, _CHUNK)], osem[b])
    outcp[_NCHUNK - 2].wait()
    outcp[_NCHUNK - 1].wait()


@jax.jit
def _emb_call(idx_flat, tok_emb, pos2d):
    mesh = plsc.VectorSubcoreMesh(core_axis_name="c", subcore_axis_name="s")
    kfn = pl.kernel(
        _emb_body,
        mesh=mesh,
        out_type=jax.ShapeDtypeStruct((_ROWS, _NE), jnp.float32),
        scratch_types=[
            pltpu.VMEM((_ROWS_PER_W,), jnp.int32),
            pltpu.VMEM((_ROWS_PER_W + _LANES,), jnp.float32),
            pltpu.VMEM((_CHUNK, _NE), jnp.float32),
            pltpu.VMEM((_CHUNK, _NE), jnp.float32),
            pltpu.VMEM((_CHUNK, _NE), jnp.float32),
            pltpu.VMEM((_CHUNK, _NE), jnp.float32),
            pltpu.SemaphoreType.DMA,
            pltpu.SemaphoreType.DMA,
            pltpu.SemaphoreType.DMA,
            pltpu.SemaphoreType.DMA,
            pltpu.SemaphoreType.DMA,
            pltpu.SemaphoreType.DMA,
        ],
    )
    return kfn(idx_flat, tok_emb, pos2d)


def kernel(idx, tok_emb, pos_emb):
    b, t = idx.shape
    out = _emb_call(idx.reshape(-1), tok_emb, pos_emb.reshape(-1, _NE)[:t])
    return out.reshape(b, t, _NE)


# speedup vs baseline: 4.3926x; 1.0376x over previous
import functools

import jax
import jax.numpy as jnp
from jax import lax
from jax.experimental import pallas as pl
from jax.experimental.pallas import tpu as pltpu
from jax.experimental.pallas import tpu_sc as plsc

_VOCAB = 100000
_NE = 768
_T = 2048
_LANES = 16
_NC = 2
_NS = 16
_NW = _NC * _NS
_ROWS = 8192
_ROWS_PER_W = _ROWS // _NW
_CHUNK = 32
_NCHUNK = _ROWS_PER_W // _CHUNK


def _emb_body(idx_hbm, tok_hbm, pos_hbm, out_hbm,
              idx_v, mask_v, rows0, rows1, pos0, pos1,
              gs0, gs1, ps0, ps1, os0, os1):
    rows = (rows0, rows1)
    pos = (pos0, pos1)
    gsem = (gs0, gs1)
    psem = (ps0, ps1)
    osem = (os0, os1)

    c = lax.axis_index("c")
    s = lax.axis_index("s")
    wid = s * _NC + c
    base = wid * _ROWS_PER_W
    t0 = base % _T

    pltpu.sync_copy(idx_hbm.at[pl.ds(base, _ROWS_PER_W)], idx_v)

    for j in range(_ROWS_PER_W // _LANES):
        sl = pl.ds(j * _LANES, _LANES)
        v = idx_v[sl]
        bad = jnp.logical_or(v < 1, v > _VOCAB)
        idx_v[sl] = jnp.where(bad, 0, v)
        mask_v[sl] = jnp.where(bad, 0.0, 1.0).astype(jnp.float32)

    def issue(ck):
        b = ck % 2
        g = pltpu.async_copy(
            tok_hbm.at[idx_v.at[pl.ds(ck * _CHUNK, _CHUNK)]], rows[b], gsem[b])
        p = pltpu.async_copy(
            pos_hbm.at[pl.ds(t0 + ck * _CHUNK, _CHUNK)], pos[b], psem[b])
        return g, p

    inflight = {0: issue(0)}
    outcp = {}
    for ck in range(_NCHUNK):
        b = ck % 2
        if ck + 1 < _NCHUNK:
            if ck - 1 >= 0:
                outcp[ck - 1].wait()
            inflight[ck + 1] = issue(ck + 1)
        g, p = inflight[ck]
        g.wait()
        p.wait()

        def row_body(r, carry, ck=ck, b=b):
            m = mask_v[pl.ds(ck * _CHUNK + r, _LANES)][0]

            @pl.when(m != 0.0)
            def _():
                for j in range(_NE // _LANES):
                    sl = pl.ds(j * _LANES, _LANES)
                    plsc.addupdate(pos[b].at[r, sl], rows[b][r, sl])

            return carry

        lax.fori_loop(0, _CHUNK, row_body, 0)
        outcp[ck] = pltpu.async_copy(
            pos[b], out_hbm.at[pl.ds(base + ck * _CHUNK, _CHUNK)], osem[b])
    outcp[_NCHUNK - 2].wait()
    outcp[_NCHUNK - 1].wait()


@jax.jit
def _emb_call(idx_flat, tok_emb, pos2d):
    mesh = plsc.VectorSubcoreMesh(core_axis_name="c", subcore_axis_name="s")
    kfn = pl.kernel(
        _emb_body,
        mesh=mesh,
        out_type=jax.ShapeDtypeStruct((_ROWS, _NE), jnp.float32),
        scratch_types=[
            pltpu.VMEM((_ROWS_PER_W,), jnp.int32),
            pltpu.VMEM((_ROWS_PER_W + _LANES,), jnp.float32),
            pltpu.VMEM((_CHUNK, _NE), jnp.float32),
            pltpu.VMEM((_CHUNK, _NE), jnp.float32),
            pltpu.VMEM((_CHUNK, _NE), jnp.float32),
            pltpu.VMEM((_CHUNK, _NE), jnp.float32),
            pltpu.SemaphoreType.DMA,
            pltpu.SemaphoreType.DMA,
            pltpu.SemaphoreType.DMA,
            pltpu.SemaphoreType.DMA,
            pltpu.SemaphoreType.DMA,
            pltpu.SemaphoreType.DMA,
        ],
    )
    return kfn(idx_flat, tok_emb, pos2d)


def kernel(idx, tok_emb, pos_emb):
    b, t = idx.shape
    out = _emb_call(idx.reshape(-1), tok_emb, pos_emb.reshape(-1, _NE)[:t])
    return out.reshape(b, t, _NE)
